# Initial kernel scaffold; baseline (speedup 1.0000x reference)
import jax
import jax.numpy as jnp
from jax.experimental import pallas as pl


def kernel(x, edge_index, edge_weight, W_cheb, b_cheb, bn_gamma, bn_beta, W_ih, W_hh, b_ih, b_hh, bn1_gamma, bn1_beta, W_lin, b_lin):
    raise NotImplementedError("write your pallas kernel here")


# speedup vs baseline: 17.7924x; 17.7924x over previous
import functools

import jax
import jax.numpy as jnp
from jax import lax
from jax.experimental import pallas as pl
from jax.experimental.pallas import tpu as pltpu


def _fused_body(x_ref, M_ref, bvec_ref, gam_ref, bet_ref, Wih_ref, Whh_ref,
                bl_ref, g1_ref, b1_ref, Wlin_ref, blin_ref, out_ref,
                s1_ref, s2_ref, scv_ref, shv_ref, t1_ref, t2_ref, y2_ref,
                *, T, N, F, H, Bblk, Btot):
    NF = N * F
    phase = pl.program_id(0)
    blk = pl.program_id(1)
    eps = 1e-5

    @pl.when(jnp.logical_and(phase == 0, blk == 0))
    def _init():
        s1_ref[...] = jnp.zeros_like(s1_ref)
        s2_ref[...] = jnp.zeros_like(s2_ref)
        t1_ref[...] = jnp.zeros_like(t1_ref)
        t2_ref[...] = jnp.zeros_like(t2_ref)

    def cheb(t):
        xt = x_ref[:, t, :]
        ot = jnp.dot(xt, M_ref[t], preferred_element_type=jnp.float32)
        return ot + bvec_ref[pl.ds(t, 1), :]

    @pl.when(phase == 0)
    def _p0():
        for t in range(T):
            ot = cheb(t)
            s1_ref[pl.ds(t, 1), :] += jnp.sum(ot, axis=0, keepdims=True)
            s2_ref[pl.ds(t, 1), :] += jnp.sum(ot * ot, axis=0, keepdims=True)

    @pl.when(jnp.logical_and(phase == 1, blk == 0))
    def _finalize_bn():
        r5 = lax.broadcasted_iota(jnp.int32, (F, NF), 0)
        c5 = lax.broadcasted_iota(jnp.int32, (F, NF), 1) % F
        G2 = (r5 == c5).astype(jnp.float32)
        rG = lax.broadcasted_iota(jnp.int32, (NF, F), 0) % F
        cG = lax.broadcasted_iota(jnp.int32, (NF, F), 1)
        G = (rG == cG).astype(jnp.float32)
        cnt = float(Btot * N)
        fsum = jnp.dot(s1_ref[...], G, preferred_element_type=jnp.float32)
        fsq = jnp.dot(s2_ref[...], G, preferred_element_type=jnp.float32)
        mean = fsum[:T] / cnt
        var = fsq[:T] / cnt - mean * mean
        scale = gam_ref[...] * lax.rsqrt(var + eps)
        shift = bet_ref[...] - mean * scale
        scv_ref[pl.ds(0, T), :] = jnp.dot(scale, G2, preferred_element_type=jnp.float32)
        shv_ref[pl.ds(0, T), :] = jnp.dot(shift, G2, preferred_element_type=jnp.float32)

    @pl.when(phase == 1)
    def _p1():
        h = jnp.zeros((Bblk, H), dtype=jnp.float32)
        c = jnp.zeros((Bblk, H), dtype=jnp.float32)
        for t in range(T):
            ot = cheb(t)
            ot = ot * scv_ref[pl.ds(t, 1), :] + shv_ref[pl.ds(t, 1), :]
            gates = (jnp.dot(ot, Wih_ref[...], preferred_element_type=jnp.float32)
                     + jnp.dot(h, Whh_ref[...], preferred_element_type=jnp.float32)
                     + bl_ref[...])
            i_g = jax.nn.sigmoid(gates[:, 0:H])
            f_g = jax.nn.sigmoid(gates[:, H:2 * H])
            g_g = jnp.tanh(gates[:, 2 * H:3 * H])
            o_g = jax.nn.sigmoid(gates[:, 3 * H:4 * H])
            c = f_g * c + i_g * g_g
            h = o_g * jnp.tanh(c)
            y2_ref[pl.ds(blk * Bblk, Bblk), pl.ds(t * H, H)] = h
            t1_ref[pl.ds(t, 1), :] += jnp.sum(h, axis=0, keepdims=True)
            t2_ref[pl.ds(t, 1), :] += jnp.sum(h * h, axis=0, keepdims=True)

    @pl.when(phase == 2)
    def _p2():
        cnt2 = float(Btot * H)
        acc = jnp.zeros((Bblk, 3), dtype=jnp.float32) + blin_ref[...]
        for t in range(T):
            y2t = y2_ref[pl.ds(blk * Bblk, Bblk), pl.ds(t * H, H)]
            m = jnp.sum(t1_ref[pl.ds(t, 1), :], axis=1, keepdims=True) / cnt2
            v = jnp.sum(t2_ref[pl.ds(t, 1), :], axis=1, keepdims=True) / cnt2 - m * m
            g1 = g1_ref[pl.ds(0, 1), pl.ds(t, 1)]
            b1 = b1_ref[pl.ds(0, 1), pl.ds(t, 1)]
            sc_t = g1 * lax.rsqrt(v + eps)
            sh_t = b1 - m * sc_t
            yb = y2t * sc_t + sh_t
            acc = acc + jnp.dot(yb, Wlin_ref[pl.ds(t * H, H), :],
                                preferred_element_type=jnp.float32)
        out_ref[...] = acc


def kernel(x, edge_index, edge_weight, W_cheb, b_cheb, bn_gamma, bn_beta,
           W_ih, W_hh, b_ih, b_hh, bn1_gamma, bn1_beta, W_lin, b_lin):
    B, T, N, F = x.shape
    NF = N * F
    H = W_hh.shape[1]
    K = W_cheb.shape[1]
    del K

    row, col = edge_index[0], edge_index[1]
    deg = jnp.zeros((N,), x.dtype).at[row].add(edge_weight)
    adj = jnp.zeros((N, N), x.dtype).at[col, row].add(edge_weight)
    dinv = jnp.where(deg > 0, lax.rsqrt(deg), 0.0)
    L = -(dinv[:, None] * adj * dinv[None, :])
    eye = jnp.eye(N, dtype=x.dtype)
    A2 = 2.0 * (L @ L) - eye
    M = (jnp.einsum('nm,tfg->tmfng', eye, W_cheb[:, 0])
         + jnp.einsum('nm,tfg->tmfng', L, W_cheb[:, 1])
         + jnp.einsum('nm,tfg->tmfng', A2, W_cheb[:, 2])).reshape(T, NF, NF)
    bvec = jnp.tile(b_cheb, (1, N))

    x_r = x.reshape(B, T, NF)
    Wih_t = W_ih.T
    Whh_t = W_hh.T
    bl = (b_ih + b_hh).reshape(1, 4 * H)
    Wlin_t = W_lin.T
    blin = b_lin.reshape(1, 3)
    g1 = bn1_gamma.reshape(1, T)
    b1 = bn1_beta.reshape(1, T)

    Bblk = 512
    NB = B // Bblk

    body = functools.partial(_fused_body, T=T, N=N, F=F, H=H,
                             Bblk=Bblk, Btot=B)

    out = pl.pallas_call(
        body,
        grid=(3, NB),
        in_specs=[
            pl.BlockSpec((Bblk, T, NF),
                         lambda p, b: (jnp.where(p == 2, NB - 1, b), 0, 0)),
            pl.BlockSpec((T, NF, NF), lambda p, b: (0, 0, 0)),
            pl.BlockSpec((T, NF), lambda p, b: (0, 0)),
            pl.BlockSpec((T, F), lambda p, b: (0, 0)),
            pl.BlockSpec((T, F), lambda p, b: (0, 0)),
            pl.BlockSpec((NF, 4 * H), lambda p, b: (0, 0)),
            pl.BlockSpec((H, 4 * H), lambda p, b: (0, 0)),
            pl.BlockSpec((1, 4 * H), lambda p, b: (0, 0)),
            pl.BlockSpec((1, T), lambda p, b: (0, 0)),
            pl.BlockSpec((1, T), lambda p, b: (0, 0)),
            pl.BlockSpec((T * H, 3), lambda p, b: (0, 0)),
            pl.BlockSpec((1, 3), lambda p, b: (0, 0)),
        ],
        out_specs=pl.BlockSpec((Bblk, 3), lambda p, b: (b, 0)),
        out_shape=jax.ShapeDtypeStruct((B, 3), jnp.float32),
        scratch_shapes=[
            pltpu.VMEM((8, NF), jnp.float32),
            pltpu.VMEM((8, NF), jnp.float32),
            pltpu.VMEM((8, NF), jnp.float32),
            pltpu.VMEM((8, NF), jnp.float32),
            pltpu.VMEM((8, H), jnp.float32),
            pltpu.VMEM((8, H), jnp.float32),
            pltpu.VMEM((B, T * H), jnp.float32),
        ],
        compiler_params=pltpu.CompilerParams(
            dimension_semantics=("arbitrary", "arbitrary"),
        ),
    )(x_r, M, bvec, bn_gamma, bn_beta, Wih_t, Whh_t, bl, g1, b1, Wlin_t, blin)
    return out


# speedup vs baseline: 18.2581x; 1.0262x over previous
import functools

import jax
import jax.numpy as jnp
from jax import lax
from jax.experimental import pallas as pl
from jax.experimental.pallas import tpu as pltpu


def _fused_body(x_ref, M_ref, bvec_ref, gam_ref, bet_ref, Wih_ref, Whh_ref,
                bl_ref, g1_ref, b1_ref, Wlin_ref, blin_ref, out_ref,
                s1_ref, s2_ref, scv_ref, shv_ref, t1_ref, t2_ref, y2_ref,
                ch_ref, *, T, N, F, H, Bblk, Btot):
    NF = N * F
    phase = pl.program_id(0)
    blk = pl.program_id(1)
    eps = 1e-5

    @pl.when(jnp.logical_and(phase == 0, blk == 0))
    def _init():
        s1_ref[...] = jnp.zeros_like(s1_ref)
        s2_ref[...] = jnp.zeros_like(s2_ref)
        t1_ref[...] = jnp.zeros_like(t1_ref)
        t2_ref[...] = jnp.zeros_like(t2_ref)

    def cheb(t):
        xt = x_ref[:, t, :]
        ot = jnp.dot(xt, M_ref[t], preferred_element_type=jnp.float32)
        return ot + bvec_ref[pl.ds(t, 1), :]

    @pl.when(phase == 0)
    def _p0():
        for t in range(T):
            ot = cheb(t)
            ch_ref[pl.ds(blk * Bblk, Bblk), pl.ds(t * NF, NF)] = ot
            s1_ref[pl.ds(t, 1), :] += jnp.sum(ot, axis=0, keepdims=True)
            s2_ref[pl.ds(t, 1), :] += jnp.sum(ot * ot, axis=0, keepdims=True)

    @pl.when(jnp.logical_and(phase == 1, blk == 0))
    def _finalize_bn():
        r5 = lax.broadcasted_iota(jnp.int32, (F, NF), 0)
        c5 = lax.broadcasted_iota(jnp.int32, (F, NF), 1) % F
        G2 = (r5 == c5).astype(jnp.float32)
        rG = lax.broadcasted_iota(jnp.int32, (NF, F), 0) % F
        cG = lax.broadcasted_iota(jnp.int32, (NF, F), 1)
        G = (rG == cG).astype(jnp.float32)
        cnt = float(Btot * N)
        fsum = jnp.dot(s1_ref[...], G, preferred_element_type=jnp.float32)
        fsq = jnp.dot(s2_ref[...], G, preferred_element_type=jnp.float32)
        mean = fsum[:T] / cnt
        var = fsq[:T] / cnt - mean * mean
        scale = gam_ref[...] * lax.rsqrt(var + eps)
        shift = bet_ref[...] - mean * scale
        scv_ref[pl.ds(0, T), :] = jnp.dot(scale, G2, preferred_element_type=jnp.float32)
        shv_ref[pl.ds(0, T), :] = jnp.dot(shift, G2, preferred_element_type=jnp.float32)

    @pl.when(phase == 1)
    def _p1():
        h = jnp.zeros((Bblk, H), dtype=jnp.float32)
        c = jnp.zeros((Bblk, H), dtype=jnp.float32)
        for t in range(T):
            ot = ch_ref[pl.ds(blk * Bblk, Bblk), pl.ds(t * NF, NF)]
            ot = ot * scv_ref[pl.ds(t, 1), :] + shv_ref[pl.ds(t, 1), :]
            gates = (jnp.dot(ot, Wih_ref[...], preferred_element_type=jnp.float32)
                     + jnp.dot(h, Whh_ref[...], preferred_element_type=jnp.float32)
                     + bl_ref[...])
            i_g = jax.nn.sigmoid(gates[:, 0:H])
            f_g = jax.nn.sigmoid(gates[:, H:2 * H])
            g_g = jnp.tanh(gates[:, 2 * H:3 * H])
            o_g = jax.nn.sigmoid(gates[:, 3 * H:4 * H])
            c = f_g * c + i_g * g_g
            h = o_g * jnp.tanh(c)
            y2_ref[pl.ds(blk * Bblk, Bblk), pl.ds(t * H, H)] = h
            t1_ref[pl.ds(t, 1), :] += jnp.sum(h, axis=0, keepdims=True)
            t2_ref[pl.ds(t, 1), :] += jnp.sum(h * h, axis=0, keepdims=True)

    @pl.when(phase == 2)
    def _p2():
        cnt2 = float(Btot * H)
        acc = jnp.zeros((Bblk, 3), dtype=jnp.float32) + blin_ref[...]
        for t in range(T):
            y2t = y2_ref[pl.ds(blk * Bblk, Bblk), pl.ds(t * H, H)]
            m = jnp.sum(t1_ref[pl.ds(t, 1), :], axis=1, keepdims=True) / cnt2
            v = jnp.sum(t2_ref[pl.ds(t, 1), :], axis=1, keepdims=True) / cnt2 - m * m
            g1 = g1_ref[pl.ds(0, 1), pl.ds(t, 1)]
            b1 = b1_ref[pl.ds(0, 1), pl.ds(t, 1)]
            sc_t = g1 * lax.rsqrt(v + eps)
            sh_t = b1 - m * sc_t
            yb = y2t * sc_t + sh_t
            acc = acc + jnp.dot(yb, Wlin_ref[pl.ds(t * H, H), :],
                                preferred_element_type=jnp.float32)
        out_ref[...] = acc


def kernel(x, edge_index, edge_weight, W_cheb, b_cheb, bn_gamma, bn_beta,
           W_ih, W_hh, b_ih, b_hh, bn1_gamma, bn1_beta, W_lin, b_lin):
    B, T, N, F = x.shape
    NF = N * F
    H = W_hh.shape[1]
    K = W_cheb.shape[1]
    del K

    row, col = edge_index[0], edge_index[1]
    deg = jnp.zeros((N,), x.dtype).at[row].add(edge_weight)
    adj = jnp.zeros((N, N), x.dtype).at[col, row].add(edge_weight)
    dinv = jnp.where(deg > 0, lax.rsqrt(deg), 0.0)
    L = -(dinv[:, None] * adj * dinv[None, :])
    eye = jnp.eye(N, dtype=x.dtype)
    A2 = 2.0 * (L @ L) - eye
    M = (jnp.einsum('nm,tfg->tmfng', eye, W_cheb[:, 0])
         + jnp.einsum('nm,tfg->tmfng', L, W_cheb[:, 1])
         + jnp.einsum('nm,tfg->tmfng', A2, W_cheb[:, 2])).reshape(T, NF, NF)
    bvec = jnp.tile(b_cheb, (1, N))

    x_r = x.reshape(B, T, NF)
    Wih_t = W_ih.T
    Whh_t = W_hh.T
    bl = (b_ih + b_hh).reshape(1, 4 * H)
    Wlin_t = W_lin.T
    blin = b_lin.reshape(1, 3)
    g1 = bn1_gamma.reshape(1, T)
    b1 = bn1_beta.reshape(1, T)

    Bblk = 512
    NB = B // Bblk

    body = functools.partial(_fused_body, T=T, N=N, F=F, H=H,
                             Bblk=Bblk, Btot=B)

    out = pl.pallas_call(
        body,
        grid=(3, NB),
        in_specs=[
            pl.BlockSpec((Bblk, T, NF),
                         lambda p, b: (jnp.where(p == 0, b, NB - 1), 0, 0)),
            pl.BlockSpec((T, NF, NF), lambda p, b: (0, 0, 0)),
            pl.BlockSpec((T, NF), lambda p, b: (0, 0)),
            pl.BlockSpec((T, F), lambda p, b: (0, 0)),
            pl.BlockSpec((T, F), lambda p, b: (0, 0)),
            pl.BlockSpec((NF, 4 * H), lambda p, b: (0, 0)),
            pl.BlockSpec((H, 4 * H), lambda p, b: (0, 0)),
            pl.BlockSpec((1, 4 * H), lambda p, b: (0, 0)),
            pl.BlockSpec((1, T), lambda p, b: (0, 0)),
            pl.BlockSpec((1, T), lambda p, b: (0, 0)),
            pl.BlockSpec((T * H, 3), lambda p, b: (0, 0)),
            pl.BlockSpec((1, 3), lambda p, b: (0, 0)),
        ],
        out_specs=pl.BlockSpec((Bblk, 3), lambda p, b: (b, 0)),
        out_shape=jax.ShapeDtypeStruct((B, 3), jnp.float32),
        scratch_shapes=[
            pltpu.VMEM((8, NF), jnp.float32),
            pltpu.VMEM((8, NF), jnp.float32),
            pltpu.VMEM((8, NF), jnp.float32),
            pltpu.VMEM((8, NF), jnp.float32),
            pltpu.VMEM((8, H), jnp.float32),
            pltpu.VMEM((8, H), jnp.float32),
            pltpu.VMEM((B, T * H), jnp.float32),
            pltpu.VMEM((B, T * NF), jnp.float32),
        ],
        compiler_params=pltpu.CompilerParams(
            dimension_semantics=("arbitrary", "arbitrary"),
        ),
    )(x_r, M, bvec, bn_gamma, bn_beta, Wih_t, Whh_t, bl, g1, b1, Wlin_t, blin)
    return out


# speedup vs baseline: 22.4808x; 1.2313x over previous
import functools

import jax
import jax.numpy as jnp
from jax import lax
from jax.experimental import pallas as pl
from jax.experimental.pallas import tpu as pltpu
from jax.experimental.pallas import tpu_sc as plsc

_NPAD = 3872
_DUMP = 3856


def _graph_scatter_body(idx_hbm, w_hbm, adj_hbm, idx_v, w_v, adj_sp, zero_v):
    cid = lax.axis_index("c")
    sid = lax.axis_index("s")

    @pl.when(jnp.logical_and(cid == 0, sid == 0))
    def _():
        pltpu.sync_copy(idx_hbm, idx_v)
        pltpu.sync_copy(w_hbm, w_v)
        z16 = jnp.zeros((16,), jnp.float32)
        for i in range(_NPAD // 16):
            zero_v[pl.ds(i * 16, 16)] = z16
        pltpu.sync_copy(zero_v, adj_sp)
        for j in range(8):
            pltpu.sync_copy(w_v.at[j], adj_sp.at[idx_v.at[j]], add=True)
        pltpu.sync_copy(adj_sp, adj_hbm)


def _build_adj_sc(edge_index, edge_weight):
    E = edge_weight.shape[0]
    N = 62
    row = edge_index[0]
    col = edge_index[1]
    idx = col * N + row
    pad = 8 * 128 - E
    idx_p = jnp.concatenate(
        [idx, jnp.full((pad,), _DUMP, jnp.int32)]).reshape(8, 128)
    w_p = jnp.concatenate(
        [edge_weight, jnp.zeros((pad,), jnp.float32)]).reshape(8, 128)
    mesh = plsc.VectorSubcoreMesh(core_axis_name="c", subcore_axis_name="s")
    adj_flat = pl.kernel(
        _graph_scatter_body,
        out_type=jax.ShapeDtypeStruct((_NPAD,), jnp.float32),
        mesh=mesh,
        scratch_types=[
            pltpu.VMEM((8, 128), jnp.int32),
            pltpu.VMEM((8, 128), jnp.float32),
            pltpu.VMEM_SHARED((_NPAD,), jnp.float32),
            pltpu.VMEM((_NPAD,), jnp.float32),
        ],
    )(idx_p, w_p)
    return adj_flat[:N * N].reshape(N, N)


def _fused_body(x_ref, M_ref, bvec_ref, gam_ref, bet_ref, Wih_ref, Whh_ref,
                bl_ref, g1_ref, b1_ref, Wlin_ref, blin_ref, out_ref,
                s1_ref, s2_ref, scv_ref, shv_ref, t1_ref, t2_ref, y2_ref,
                ch_ref, *, T, N, F, H, Bblk, Btot):
    NF = N * F
    phase = pl.program_id(0)
    blk = pl.program_id(1)
    eps = 1e-5

    @pl.when(jnp.logical_and(phase == 0, blk == 0))
    def _init():
        s1_ref[...] = jnp.zeros_like(s1_ref)
        s2_ref[...] = jnp.zeros_like(s2_ref)
        t1_ref[...] = jnp.zeros_like(t1_ref)
        t2_ref[...] = jnp.zeros_like(t2_ref)

    def cheb(t):
        xt = x_ref[:, t, :]
        ot = jnp.dot(xt, M_ref[t], preferred_element_type=jnp.float32)
        return ot + bvec_ref[pl.ds(t, 1), :]

    @pl.when(phase == 0)
    def _p0():
        for t in range(T):
            ot = cheb(t)
            ch_ref[pl.ds(blk * Bblk, Bblk), pl.ds(t * NF, NF)] = ot
            s1_ref[pl.ds(t, 1), :] += jnp.sum(ot, axis=0, keepdims=True)
            s2_ref[pl.ds(t, 1), :] += jnp.sum(ot * ot, axis=0, keepdims=True)

    @pl.when(jnp.logical_and(phase == 1, blk == 0))
    def _finalize_bn():
        r5 = lax.broadcasted_iota(jnp.int32, (F, NF), 0)
        c5 = lax.broadcasted_iota(jnp.int32, (F, NF), 1) % F
        G2 = (r5 == c5).astype(jnp.float32)
        rG = lax.broadcasted_iota(jnp.int32, (NF, F), 0) % F
        cG = lax.broadcasted_iota(jnp.int32, (NF, F), 1)
        G = (rG == cG).astype(jnp.float32)
        cnt = float(Btot * N)
        fsum = jnp.dot(s1_ref[...], G, preferred_element_type=jnp.float32)
        fsq = jnp.dot(s2_ref[...], G, preferred_element_type=jnp.float32)
        mean = fsum[:T] / cnt
        var = fsq[:T] / cnt - mean * mean
        scale = gam_ref[...] * lax.rsqrt(var + eps)
        shift = bet_ref[...] - mean * scale
        scv_ref[pl.ds(0, T), :] = jnp.dot(scale, G2, preferred_element_type=jnp.float32)
        shv_ref[pl.ds(0, T), :] = jnp.dot(shift, G2, preferred_element_type=jnp.float32)

    @pl.when(phase == 1)
    def _p1():
        h = jnp.zeros((Bblk, H), dtype=jnp.float32)
        c = jnp.zeros((Bblk, H), dtype=jnp.float32)
        for t in range(T):
            ot = ch_ref[pl.ds(blk * Bblk, Bblk), pl.ds(t * NF, NF)]
            ot = ot * scv_ref[pl.ds(t, 1), :] + shv_ref[pl.ds(t, 1), :]
            gates = (jnp.dot(ot, Wih_ref[...], preferred_element_type=jnp.float32)
                     + jnp.dot(h, Whh_ref[...], preferred_element_type=jnp.float32)
                     + bl_ref[...])
            i_g = jax.nn.sigmoid(gates[:, 0:H])
            f_g = jax.nn.sigmoid(gates[:, H:2 * H])
            g_g = jnp.tanh(gates[:, 2 * H:3 * H])
            o_g = jax.nn.sigmoid(gates[:, 3 * H:4 * H])
            c = f_g * c + i_g * g_g
            h = o_g * jnp.tanh(c)
            y2_ref[pl.ds(blk * Bblk, Bblk), pl.ds(t * H, H)] = h
            t1_ref[pl.ds(t, 1), :] += jnp.sum(h, axis=0, keepdims=True)
            t2_ref[pl.ds(t, 1), :] += jnp.sum(h * h, axis=0, keepdims=True)

    @pl.when(phase == 2)
    def _p2():
        cnt2 = float(Btot * H)
        acc = jnp.zeros((Bblk, 3), dtype=jnp.float32) + blin_ref[...]
        for t in range(T):
            y2t = y2_ref[pl.ds(blk * Bblk, Bblk), pl.ds(t * H, H)]
            m = jnp.sum(t1_ref[pl.ds(t, 1), :], axis=1, keepdims=True) / cnt2
            v = jnp.sum(t2_ref[pl.ds(t, 1), :], axis=1, keepdims=True) / cnt2 - m * m
            g1 = g1_ref[pl.ds(0, 1), pl.ds(t, 1)]
            b1 = b1_ref[pl.ds(0, 1), pl.ds(t, 1)]
            sc_t = g1 * lax.rsqrt(v + eps)
            sh_t = b1 - m * sc_t
            yb = y2t * sc_t + sh_t
            acc = acc + jnp.dot(yb, Wlin_ref[pl.ds(t * H, H), :],
                                preferred_element_type=jnp.float32)
        out_ref[...] = acc


def kernel(x, edge_index, edge_weight, W_cheb, b_cheb, bn_gamma, bn_beta,
           W_ih, W_hh, b_ih, b_hh, bn1_gamma, bn1_beta, W_lin, b_lin):
    B, T, N, F = x.shape
    NF = N * F
    H = W_hh.shape[1]
    K = W_cheb.shape[1]
    del K

    adj = _build_adj_sc(edge_index, edge_weight)
    deg = adj.sum(axis=0)
    dinv = jnp.where(deg > 0, lax.rsqrt(deg), 0.0)
    L = -(dinv[:, None] * adj * dinv[None, :])
    eye = jnp.eye(N, dtype=x.dtype)
    A2 = 2.0 * (L @ L) - eye
    M = (jnp.einsum('nm,tfg->tmfng', eye, W_cheb[:, 0])
         + jnp.einsum('nm,tfg->tmfng', L, W_cheb[:, 1])
         + jnp.einsum('nm,tfg->tmfng', A2, W_cheb[:, 2])).reshape(T, NF, NF)
    bvec = jnp.tile(b_cheb, (1, N))

    x_r = x.reshape(B, T, NF)
    Wih_t = W_ih.T
    Whh_t = W_hh.T
    bl = (b_ih + b_hh).reshape(1, 4 * H)
    Wlin_t = W_lin.T
    blin = b_lin.reshape(1, 3)
    g1 = bn1_gamma.reshape(1, T)
    b1 = bn1_beta.reshape(1, T)

    Bblk = 512
    NB = B // Bblk

    body = functools.partial(_fused_body, T=T, N=N, F=F, H=H,
                             Bblk=Bblk, Btot=B)

    out = pl.pallas_call(
        body,
        grid=(3, NB),
        in_specs=[
            pl.BlockSpec((Bblk, T, NF),
                         lambda p, b: (jnp.where(p == 0, b, NB - 1), 0, 0)),
            pl.BlockSpec((T, NF, NF), lambda p, b: (0, 0, 0)),
            pl.BlockSpec((T, NF), lambda p, b: (0, 0)),
            pl.BlockSpec((T, F), lambda p, b: (0, 0)),
            pl.BlockSpec((T, F), lambda p, b: (0, 0)),
            pl.BlockSpec((NF, 4 * H), lambda p, b: (0, 0)),
            pl.BlockSpec((H, 4 * H), lambda p, b: (0, 0)),
            pl.BlockSpec((1, 4 * H), lambda p, b: (0, 0)),
            pl.BlockSpec((1, T), lambda p, b: (0, 0)),
            pl.BlockSpec((1, T), lambda p, b: (0, 0)),
            pl.BlockSpec((T * H, 3), lambda p, b: (0, 0)),
            pl.BlockSpec((1, 3), lambda p, b: (0, 0)),
        ],
        out_specs=pl.BlockSpec((Bblk, 3), lambda p, b: (b, 0)),
        out_shape=jax.ShapeDtypeStruct((B, 3), jnp.float32),
        scratch_shapes=[
            pltpu.VMEM((8, NF), jnp.float32),
            pltpu.VMEM((8, NF), jnp.float32),
            pltpu.VMEM((8, NF), jnp.float32),
            pltpu.VMEM((8, NF), jnp.float32),
            pltpu.VMEM((8, H), jnp.float32),
            pltpu.VMEM((8, H), jnp.float32),
            pltpu.VMEM((B, T * H), jnp.float32),
            pltpu.VMEM((B, T * NF), jnp.float32),
        ],
        compiler_params=pltpu.CompilerParams(
            dimension_semantics=("arbitrary", "arbitrary"),
        ),
    )(x_r, M, bvec, bn_gamma, bn_beta, Wih_t, Whh_t, bl, g1, b1, Wlin_t, blin)
    return out


# speedup vs baseline: 36.3602x; 1.6174x over previous
import functools

import jax
import jax.numpy as jnp
from jax import lax
from jax.experimental import pallas as pl
from jax.experimental.pallas import tpu as pltpu
from jax.experimental.pallas import tpu_sc as plsc

_NPAD = 3872
_DUMP = 3856


def _graph_scatter_body(idx_hbm, w_hbm, adj_hbm, idx_v, w_v, adj_sp, zero_v):
    cid = lax.axis_index("c")
    sid = lax.axis_index("s")

    @pl.when(jnp.logical_and(cid == 0, sid == 0))
    def _():
        pltpu.sync_copy(idx_hbm, idx_v)
        pltpu.sync_copy(w_hbm, w_v)
        z16 = jnp.zeros((16,), jnp.float32)
        for i in range(_NPAD // 16):
            zero_v[pl.ds(i * 16, 16)] = z16
        pltpu.sync_copy(zero_v, adj_sp)
        for j in range(8):
            sl = pl.ds(j * 128, 128)
            pltpu.sync_copy(w_v.at[sl], adj_sp.at[idx_v.at[sl]], add=True)
        pltpu.sync_copy(adj_sp, adj_hbm)


def _build_adj_sc(edge_index, edge_weight):
    E = edge_weight.shape[0]
    N = 62
    idx = edge_index[1] * N + edge_index[0]
    pad = 1024 - E
    idx_p = jnp.concatenate([idx, jnp.full((pad,), _DUMP, jnp.int32)])
    w_p = jnp.concatenate([edge_weight, jnp.zeros((pad,), jnp.float32)])
    mesh = plsc.VectorSubcoreMesh(core_axis_name="c", subcore_axis_name="s")
    adj_flat = pl.kernel(
        _graph_scatter_body,
        out_type=jax.ShapeDtypeStruct((_NPAD,), jnp.float32),
        mesh=mesh,
        scratch_types=[
            pltpu.VMEM((1024,), jnp.int32),
            pltpu.VMEM((1024,), jnp.float32),
            pltpu.VMEM_SHARED((_NPAD,), jnp.float32),
            pltpu.VMEM((_NPAD,), jnp.float32),
        ],
    )(idx_p, w_p)
    return adj_flat[:N * N].reshape(N, N)


def _dot_t(a, b):
    return lax.dot_general(a, b, (((1,), (1,)), ((), ())),
                           preferred_element_type=jnp.float32)


def _fused_body(x_ref, adj_ref, Wch_ref, bvec_ref, gam_ref, bet_ref,
                Wih_ref, Whh_ref, bl_ref, g1_ref, b1_ref, Wlin_ref, blin_ref,
                out_ref, M_ref, s1_ref, s2_ref, scv_ref, shv_ref,
                t1_ref, t2_ref, y2_ref, ch_ref, *, T, N, F, H, Bblk, Btot):
    NF = N * F
    phase = pl.program_id(0)
    blk = pl.program_id(1)
    eps = 1e-5
    f32 = jnp.float32

    @pl.when(jnp.logical_and(phase == 0, blk == 0))
    def _init():
        s1_ref[...] = jnp.zeros_like(s1_ref)
        s2_ref[...] = jnp.zeros_like(s2_ref)
        t1_ref[...] = jnp.zeros_like(t1_ref)
        t2_ref[...] = jnp.zeros_like(t2_ref)
        adj = adj_ref[...]
        deg = jnp.sum(adj, axis=0, keepdims=True)
        dinv = jnp.where(deg > 0, lax.rsqrt(deg), 0.0)
        outer = lax.dot_general(dinv, dinv, (((0,), (0,)), ((), ())),
                                preferred_element_type=f32)
        L = -(adj * outer)
        rN = lax.broadcasted_iota(jnp.int32, (N, N), 0)
        cN = lax.broadcasted_iota(jnp.int32, (N, N), 1)
        eyeN = (rN == cN).astype(f32)
        A2 = 2.0 * jnp.dot(L, L, preferred_element_type=f32) - eyeN
        rE = lax.broadcasted_iota(jnp.int32, (NF, N), 0)
        cE = lax.broadcasted_iota(jnp.int32, (NF, N), 1)
        EN = (rE // F == cE).astype(f32)
        rF = lax.broadcasted_iota(jnp.int32, (NF, F), 0)
        cF = lax.broadcasted_iota(jnp.int32, (NF, F), 1)
        EF = (rF % F == cF).astype(f32)
        Qs = []
        for A in (eyeN, L, A2):
            Qs.append(_dot_t(_dot_t(EN, A), EN))
        for t in range(T):
            acc = jnp.zeros((NF, NF), dtype=f32)
            for k in range(3):
                W = Wch_ref[t, k]
                R = _dot_t(lax.dot_general(
                    EF, W, (((1,), (0,)), ((), ())),
                    preferred_element_type=f32), EF)
                acc = acc + Qs[k] * R
            M_ref[pl.ds(t * NF, NF), :] = acc

    def cheb(t):
        xt = x_ref[:, t, :]
        ot = jnp.dot(xt, M_ref[pl.ds(t * NF, NF), :],
                     preferred_element_type=f32)
        return ot + bvec_ref[pl.ds(t, 1), :]

    @pl.when(phase == 0)
    def _p0():
        for t in range(T):
            ot = cheb(t)
            ch_ref[pl.ds(blk * Bblk, Bblk), pl.ds(t * NF, NF)] = ot
            s1_ref[pl.ds(t, 1), :] += jnp.sum(ot, axis=0, keepdims=True)
            s2_ref[pl.ds(t, 1), :] += jnp.sum(ot * ot, axis=0, keepdims=True)

    @pl.when(jnp.logical_and(phase == 1, blk == 0))
    def _finalize_bn():
        r5 = lax.broadcasted_iota(jnp.int32, (F, NF), 0)
        c5 = lax.broadcasted_iota(jnp.int32, (F, NF), 1) % F
        G2 = (r5 == c5).astype(f32)
        rG = lax.broadcasted_iota(jnp.int32, (NF, F), 0) % F
        cG = lax.broadcasted_iota(jnp.int32, (NF, F), 1)
        G = (rG == cG).astype(f32)
        cnt = float(Btot * N)
        fsum = jnp.dot(s1_ref[...], G, preferred_element_type=f32)
        fsq = jnp.dot(s2_ref[...], G, preferred_element_type=f32)
        mean = fsum[:T] / cnt
        var = fsq[:T] / cnt - mean * mean
        scale = gam_ref[...] * lax.rsqrt(var + eps)
        shift = bet_ref[...] - mean * scale
        scv_ref[pl.ds(0, T), :] = jnp.dot(scale, G2, preferred_element_type=f32)
        shv_ref[pl.ds(0, T), :] = jnp.dot(shift, G2, preferred_element_type=f32)

    @pl.when(phase == 1)
    def _p1():
        h = jnp.zeros((Bblk, H), dtype=f32)
        c = jnp.zeros((Bblk, H), dtype=f32)
        for t in range(T):
            ot = ch_ref[pl.ds(blk * Bblk, Bblk), pl.ds(t * NF, NF)]
            ot = ot * scv_ref[pl.ds(t, 1), :] + shv_ref[pl.ds(t, 1), :]
            gates = (_dot_t(ot, Wih_ref[...]) + _dot_t(h, Whh_ref[...])
                     + bl_ref[...])
            i_g = jax.nn.sigmoid(gates[:, 0:H])
            f_g = jax.nn.sigmoid(gates[:, H:2 * H])
            g_g = jnp.tanh(gates[:, 2 * H:3 * H])
            o_g = jax.nn.sigmoid(gates[:, 3 * H:4 * H])
            c = f_g * c + i_g * g_g
            h = o_g * jnp.tanh(c)
            y2_ref[pl.ds(blk * Bblk, Bblk), pl.ds(t * H, H)] = h
            t1_ref[pl.ds(t, 1), :] += jnp.sum(h, axis=0, keepdims=True)
            t2_ref[pl.ds(t, 1), :] += jnp.sum(h * h, axis=0, keepdims=True)

    @pl.when(phase == 2)
    def _p2():
        cnt2 = float(Btot * H)
        acc = jnp.zeros((Bblk, 3), dtype=f32) + blin_ref[...]
        for t in range(T):
            y2t = y2_ref[pl.ds(blk * Bblk, Bblk), pl.ds(t * H, H)]
            m = jnp.sum(t1_ref[pl.ds(t, 1), :], axis=1, keepdims=True) / cnt2
            v = jnp.sum(t2_ref[pl.ds(t, 1), :], axis=1, keepdims=True) / cnt2 - m * m
            g1 = g1_ref[pl.ds(0, 1), pl.ds(t, 1)]
            b1 = b1_ref[pl.ds(0, 1), pl.ds(t, 1)]
            sc_t = g1 * lax.rsqrt(v + eps)
            sh_t = b1 - m * sc_t
            yb = y2t * sc_t + sh_t
            acc = acc + _dot_t(yb, Wlin_ref[:, pl.ds(t * H, H)])
        out_ref[...] = acc


def kernel(x, edge_index, edge_weight, W_cheb, b_cheb, bn_gamma, bn_beta,
           W_ih, W_hh, b_ih, b_hh, bn1_gamma, bn1_beta, W_lin, b_lin):
    B, T, N, F = x.shape
    NF = N * F
    H = W_hh.shape[1]

    adj = _build_adj_sc(edge_index, edge_weight)

    bvec = jnp.tile(b_cheb, (1, N))
    x_r = x.reshape(B, T, NF)
    bl = (b_ih + b_hh).reshape(1, 4 * H)
    blin = b_lin.reshape(1, 3)
    g1 = bn1_gamma.reshape(1, T)
    b1 = bn1_beta.reshape(1, T)

    Bblk = 512
    NB = B // Bblk

    body = functools.partial(_fused_body, T=T, N=N, F=F, H=H,
                             Bblk=Bblk, Btot=B)

    out = pl.pallas_call(
        body,
        grid=(3, NB),
        in_specs=[
            pl.BlockSpec((Bblk, T, NF),
                         lambda p, b: (jnp.where(p == 0, b, NB - 1), 0, 0)),
            pl.BlockSpec((N, N), lambda p, b: (0, 0)),
            pl.BlockSpec((T, 3, F, F), lambda p, b: (0, 0, 0, 0)),
            pl.BlockSpec((T, NF), lambda p, b: (0, 0)),
            pl.BlockSpec((T, F), lambda p, b: (0, 0)),
            pl.BlockSpec((T, F), lambda p, b: (0, 0)),
            pl.BlockSpec((4 * H, NF), lambda p, b: (0, 0)),
            pl.BlockSpec((4 * H, H), lambda p, b: (0, 0)),
            pl.BlockSpec((1, 4 * H), lambda p, b: (0, 0)),
            pl.BlockSpec((1, T), lambda p, b: (0, 0)),
            pl.BlockSpec((1, T), lambda p, b: (0, 0)),
            pl.BlockSpec((3, T * H), lambda p, b: (0, 0)),
            pl.BlockSpec((1, 3), lambda p, b: (0, 0)),
        ],
        out_specs=pl.BlockSpec((Bblk, 3), lambda p, b: (b, 0)),
        out_shape=jax.ShapeDtypeStruct((B, 3), jnp.float32),
        scratch_shapes=[
            pltpu.VMEM((T * NF, NF), jnp.float32),
            pltpu.VMEM((8, NF), jnp.float32),
            pltpu.VMEM((8, NF), jnp.float32),
            pltpu.VMEM((8, NF), jnp.float32),
            pltpu.VMEM((8, NF), jnp.float32),
            pltpu.VMEM((8, H), jnp.float32),
            pltpu.VMEM((8, H), jnp.float32),
            pltpu.VMEM((B, T * H), jnp.float32),
            pltpu.VMEM((B, T * NF), jnp.float32),
        ],
        compiler_params=pltpu.CompilerParams(
            dimension_semantics=("arbitrary", "arbitrary"),
        ),
    )(x_r, adj, W_cheb, bvec, bn_gamma, bn_beta, W_ih, W_hh, bl, g1, b1,
      W_lin, blin)
    return out
